# Initial kernel scaffold; baseline (speedup 1.0000x reference)
import jax
import jax.numpy as jnp
from jax.experimental import pallas as pl


def kernel(features, pseudo_labels):
    raise NotImplementedError("write your pallas kernel here")


# speedup vs baseline: 1.6027x; 1.6027x over previous
import jax
import jax.numpy as jnp
from jax import lax
from jax.experimental import pallas as pl
from jax.experimental.pallas import tpu as pltpu
from jax.experimental.pallas import tpu_sc as plsc

N_CLS = 3
D = 512
B = 16384

NC = 2
NS = 16
L = 16

COLS = D // NC
ROWS = B // NS
BLK = 128
NBLK = ROWS // BLK
CHUNKS = COLS // L
GROUPS = ROWS // L


def _body(feat_hbm, lab_hbm, out_hbm,
          lab_v, off_v, cbuf_v, tbuf_v, fbuf0, fbuf1, rbuf_v,
          sem0, sem1, shared):
  cid = lax.axis_index("c")
  sid = lax.axis_index("s")
  row0 = sid * ROWS
  col0 = cid * COLS

  iota = lax.iota(jnp.int32, L)
  zf = jnp.zeros((L,), jnp.float32)

  def zero_body(r, _):
    for j in range(CHUNKS):
      cbuf_v[r, pl.ds(j * L, L)] = zf
    return 0
  lax.fori_loop(0, 4, zero_body, 0)

  pltpu.sync_copy(lab_hbm.at[pl.ds(row0 * N_CLS, ROWS * N_CLS)], lab_v)

  iota3 = iota * 3

  def cls_body(g, carry):
    c0, c1, c2 = carry
    base = g * (L * N_CLS)
    p0 = plsc.load_gather(lab_v, [iota3 + base])
    p1 = plsc.load_gather(lab_v, [iota3 + (base + 1)])
    p2 = plsc.load_gather(lab_v, [iota3 + (base + 2)])
    cls = jnp.where(p1 > p0, 1, 0).astype(jnp.int32)
    m01 = jnp.maximum(p0, p1)
    cls = jnp.where(p2 > m01, 2, cls)
    off_v[pl.ds(g * L, L)] = cls
    one = jnp.float32(1.0)
    zero = jnp.float32(0.0)
    c0 = c0 + jnp.where(cls == 0, one, zero)
    c1 = c1 + jnp.where(cls == 1, one, zero)
    c2 = c2 + jnp.where(cls == 2, one, zero)
    return (c0, c1, c2)

  c0, c1, c2 = lax.fori_loop(0, GROUPS, cls_body, (zf, zf, zf))

  fbufs = (fbuf0, fbuf1)
  sems = (sem0, sem1)

  def mk_copy(blk, par):
    return pltpu.make_async_copy(
        feat_hbm.at[pl.ds(row0 + blk * BLK, BLK), pl.ds(col0, COLS)],
        fbufs[par], sems[par])

  mk_copy(0, 0).start()
  mk_copy(1, 1).start()

  zi = jnp.zeros((L,), jnp.int32)
  dnums = lax.GatherDimensionNumbers(
      offset_dims=(), collapsed_slice_dims=(0,), start_index_map=(0,))

  def splat(vec, lane):
    return lax.gather(vec, (zi + lane)[:, None], dnums, slice_sizes=(1,),
                      mode=lax.GatherScatterMode.PROMISE_IN_BOUNDS)

  def process(blk, fbuf):
    def row_body(r, _):
      grp = r & ~(L - 1)
      lane = r & (L - 1)
      cls_vec = off_v[pl.ds(blk * BLK + grp, L)]
      cls_splat = splat(cls_vec, lane)
      for j in range(CHUNKS):
        x = fbuf[r, pl.ds(j * L, L)]
        plsc.addupdate_scatter(cbuf_v, [cls_splat, iota + j * L], x)
      return 0
    lax.fori_loop(0, BLK, row_body, 0)

  for blk in range(NBLK):
    par = blk % 2
    mk_copy(blk, par).wait()
    process(blk, fbufs[par])
    if blk + 2 < NBLK:
      mk_copy(blk + 2, par).start()

  cbuf_v[3, pl.ds(0, L)] = c0
  cbuf_v[3, pl.ds(L, L)] = c1
  cbuf_v[3, pl.ds(2 * L, L)] = c2

  pltpu.sync_copy(cbuf_v, shared.at[sid])
  plsc.subcore_barrier()

  @pl.when(sid == 0)
  def _finalize():
    pltpu.sync_copy(shared.at[0], rbuf_v)

    def red_body(t, _):
      pltpu.sync_copy(shared.at[t], tbuf_v)
      for k in range(4):
        for j in range(CHUNKS):
          rbuf_v[k, pl.ds(j * L, L)] = (
              rbuf_v[k, pl.ds(j * L, L)] + tbuf_v[k, pl.ds(j * L, L)])
      return 0
    lax.fori_loop(1, NS, red_body, 0)

    ones = jnp.ones((L,), jnp.float32)
    for k in range(N_CLS):
      cvec = rbuf_v[3, pl.ds(k * L, L)]
      cnt = plsc.cumsum(cvec)[L - 1] * ones
      safe = jnp.where(cnt > 0, cnt, ones)
      scale = jnp.where(cnt > 0, ones / safe, ones)
      for j in range(CHUNKS):
        rbuf_v[k, pl.ds(j * L, L)] = rbuf_v[k, pl.ds(j * L, L)] * scale
    pltpu.sync_copy(rbuf_v.at[pl.ds(0, N_CLS)],
                    out_hbm.at[:, pl.ds(col0, COLS)])


@jax.jit
def kernel(features, pseudo_labels):
  mesh = plsc.VectorSubcoreMesh(core_axis_name="c", subcore_axis_name="s")
  run = pl.kernel(
      _body,
      out_type=jax.ShapeDtypeStruct((N_CLS, D), jnp.float32),
      mesh=mesh,
      compiler_params=pltpu.CompilerParams(needs_layout_passes=False),
      scratch_types=[
          pltpu.VMEM((ROWS * N_CLS,), jnp.float32),
          pltpu.VMEM((ROWS,), jnp.int32),
          pltpu.VMEM((4, COLS), jnp.float32),
          pltpu.VMEM((4, COLS), jnp.float32),
          pltpu.VMEM((BLK, COLS), jnp.float32),
          pltpu.VMEM((BLK, COLS), jnp.float32),
          pltpu.VMEM((4, COLS), jnp.float32),
          pltpu.SemaphoreType.DMA,
          pltpu.SemaphoreType.DMA,
          pltpu.VMEM_SHARED((NS, 4, COLS), jnp.float32),
      ],
  )
  return run(features, pseudo_labels.reshape(-1))


# speedup vs baseline: 1.6918x; 1.0556x over previous
import jax
import jax.numpy as jnp
from jax import lax
from jax.experimental import pallas as pl
from jax.experimental.pallas import tpu as pltpu
from jax.experimental.pallas import tpu_sc as plsc

N_CLS = 3
D = 512
B = 16384

NC = 2
NS = 16
L = 16

COLS = D // NC
ROWS = B // NS
BLK = 128
NBLK = ROWS // BLK
CHUNKS = COLS // L
GROUPS = ROWS // L
SROWS = 8


def _body(feat_hbm, lab_hbm, out_hbm,
          lab_v, off_v, tot_v, cbuf_v, tbuf_v, rbuf_v, fbuf0, fbuf1,
          sem0, sem1, shared):
  cid = lax.axis_index("c")
  sid = lax.axis_index("s")
  row0 = sid * ROWS
  col0 = cid * COLS

  iota = lax.iota(jnp.int32, L)
  zf = jnp.zeros((L,), jnp.float32)
  zi = jnp.zeros((L,), jnp.int32)

  pltpu.sync_copy(lab_hbm.at[pl.ds(row0 * N_CLS, ROWS * N_CLS)], lab_v)

  for j in range(CHUNKS):
    tot_v[pl.ds(j * L, L)] = zf

  iota3 = iota * 3

  def cls_body(g, carry):
    c0, c1, c2 = carry
    base = g * (L * N_CLS)
    p0 = plsc.load_gather(lab_v, [iota3 + base])
    p1 = plsc.load_gather(lab_v, [iota3 + (base + 1)])
    p2 = plsc.load_gather(lab_v, [iota3 + (base + 2)])
    cls = jnp.where(p1 > p0, 1, 0).astype(jnp.int32)
    m01 = jnp.maximum(p0, p1)
    cls = jnp.where(p2 > m01, 2, cls)
    off_v[pl.ds(g * L, L)] = cls
    one = jnp.float32(1.0)
    zero = jnp.float32(0.0)
    c0 = c0 + jnp.where(cls == 0, one, zero)
    c1 = c1 + jnp.where(cls == 1, one, zero)
    c2 = c2 + jnp.where(cls == 2, one, zero)
    return (c0, c1, c2)

  c0, c1, c2 = lax.fori_loop(0, GROUPS, cls_body, (zf, zf, zf))

  fbufs = (fbuf0, fbuf1)
  lsems = (sem0, sem1)

  def mk_load(blk, par):
    return pltpu.make_async_copy(
        feat_hbm.at[pl.ds(row0 + blk * BLK, BLK), pl.ds(col0, COLS)],
        fbufs[par], lsems[par])

  mk_load(0, 0).start()
  mk_load(1, 1).start()

  one = jnp.float32(1.0)
  zero = jnp.float32(0.0)

  def process(blk, fbuf, accs):
    def row_body(r, accs):
      a1, a2 = accs
      cls_splat = plsc.load_gather(off_v, [zi + (blk * BLK + r)])
      m1 = jnp.where(cls_splat == 1, one, zero)
      m2 = jnp.where(cls_splat == 2, one, zero)
      na1 = []
      na2 = []
      for j in range(CHUNKS):
        x = fbuf[r, pl.ds(j * L, L)]
        plsc.addupdate(tot_v.at[pl.ds(j * L, L)], x)
        na1.append(a1[j] + x * m1)
        na2.append(a2[j] + x * m2)
      return (tuple(na1), tuple(na2))
    return lax.fori_loop(0, BLK, row_body, accs)

  accs = (tuple(zf for _ in range(CHUNKS)), tuple(zf for _ in range(CHUNKS)))
  for blk in range(NBLK):
    par = blk % 2
    mk_load(blk, par).wait()
    accs = process(blk, fbufs[par], accs)
    if blk + 2 < NBLK:
      mk_load(blk + 2, par).start()
  a1, a2 = accs

  n0 = zf + jnp.sum(c0)
  n1 = zf + jnp.sum(c1)
  n2 = zf + jnp.sum(c2)
  for j in range(CHUNKS):
    t = tot_v[pl.ds(j * L, L)]
    cbuf_v[0, pl.ds(j * L, L)] = t - a1[j] - a2[j]
    cbuf_v[1, pl.ds(j * L, L)] = a1[j]
    cbuf_v[2, pl.ds(j * L, L)] = a2[j]
    cbuf_v[3, pl.ds(j * L, L)] = n0
    cbuf_v[4, pl.ds(j * L, L)] = n1
    cbuf_v[5, pl.ds(j * L, L)] = n2
  pltpu.sync_copy(cbuf_v, shared.at[pl.ds(sid * SROWS, SROWS)])
  plsc.subcore_barrier()

  @pl.when(sid == 0)
  def _finalize():
    pltpu.sync_copy(shared.at[pl.ds(0, SROWS)], rbuf_v)

    def red_body(t, _):
      pltpu.sync_copy(shared.at[pl.ds(t * SROWS, SROWS)], tbuf_v)
      for k in range(6):
        for j in range(CHUNKS):
          rbuf_v[k, pl.ds(j * L, L)] = (
              rbuf_v[k, pl.ds(j * L, L)] + tbuf_v[k, pl.ds(j * L, L)])
      return 0
    lax.fori_loop(1, NS, red_body, 0)

    ones = jnp.ones((L,), jnp.float32)
    for k in range(N_CLS):
      for j in range(CHUNKS):
        cnt = rbuf_v[3 + k, pl.ds(j * L, L)]
        safe = jnp.where(cnt > 0, cnt, ones)
        scale = jnp.where(cnt > 0, ones / safe, ones)
        rbuf_v[k, pl.ds(j * L, L)] = rbuf_v[k, pl.ds(j * L, L)] * scale
    pltpu.sync_copy(rbuf_v.at[pl.ds(0, N_CLS)],
                    out_hbm.at[:, pl.ds(col0, COLS)])


@jax.jit
def kernel(features, pseudo_labels):
  mesh = plsc.VectorSubcoreMesh(core_axis_name="c", subcore_axis_name="s")
  run = pl.kernel(
      _body,
      out_type=jax.ShapeDtypeStruct((N_CLS, D), jnp.float32),
      mesh=mesh,
      compiler_params=pltpu.CompilerParams(needs_layout_passes=False),
      scratch_types=[
          pltpu.VMEM((ROWS * N_CLS,), jnp.float32),
          pltpu.VMEM((ROWS,), jnp.int32),
          pltpu.VMEM((COLS,), jnp.float32),
          pltpu.VMEM((SROWS, COLS), jnp.float32),
          pltpu.VMEM((SROWS, COLS), jnp.float32),
          pltpu.VMEM((SROWS, COLS), jnp.float32),
          pltpu.VMEM((BLK, COLS), jnp.float32),
          pltpu.VMEM((BLK, COLS), jnp.float32),
          pltpu.SemaphoreType.DMA,
          pltpu.SemaphoreType.DMA,
          pltpu.VMEM_SHARED((NS * SROWS, COLS), jnp.float32),
      ],
  )
  return run(features, pseudo_labels.reshape(-1))


# speedup vs baseline: 2.5775x; 1.5235x over previous
import jax
import jax.numpy as jnp
from jax import lax
from jax.experimental import pallas as pl
from jax.experimental.pallas import tpu as pltpu
from jax.experimental.pallas import tpu_sc as plsc

N_CLS = 3
D = 512
B = 16384

NC = 2
NS = 16
L = 16

COLS = D // NC
ROWS = B // NS
BLK = 128
NBLK = ROWS // BLK
CHUNKS = COLS // L
GROUPS = ROWS // L
SROWS = 8


def _body(feat_hbm, lab_hbm, out_hbm,
          lab_v, off_v, cnt_v, cbuf_v, tbuf_v, rbuf_v, fbuf0, fbuf1,
          sem0, sem1, shared):
  cid = lax.axis_index("c")
  sid = lax.axis_index("s")
  row0 = sid * ROWS
  col0 = cid * COLS

  iota = lax.iota(jnp.int32, L)
  zf = jnp.zeros((L,), jnp.float32)
  zi = jnp.zeros((L,), jnp.int32)

  pltpu.sync_copy(lab_hbm.at[pl.ds(row0 * N_CLS, ROWS * N_CLS)], lab_v)

  for k in range(6):
    for j in range(CHUNKS):
      cbuf_v[k, pl.ds(j * L, L)] = zf
  for k in range(N_CLS):
    cnt_v[k, pl.ds(0, L)] = zf

  iota3 = iota * 3
  one = jnp.float32(1.0)
  zero = jnp.float32(0.0)

  @plsc.parallel_loop(0, GROUPS, unroll=4)
  def _cls(g):
    base = g * (L * N_CLS)
    p0 = plsc.load_gather(lab_v, [iota3 + base])
    p1 = plsc.load_gather(lab_v, [iota3 + (base + 1)])
    p2 = plsc.load_gather(lab_v, [iota3 + (base + 2)])
    cls = jnp.where(p1 > p0, 1, 0).astype(jnp.int32)
    m01 = jnp.maximum(p0, p1)
    cls = jnp.where(p2 > m01, 2, cls)
    off_v[pl.ds(g * L, L)] = cls
    plsc.addupdate(cnt_v.at[0, pl.ds(0, L)], jnp.where(cls == 0, one, zero))
    plsc.addupdate(cnt_v.at[1, pl.ds(0, L)], jnp.where(cls == 1, one, zero))
    plsc.addupdate(cnt_v.at[2, pl.ds(0, L)], jnp.where(cls == 2, one, zero))

  fbufs = (fbuf0, fbuf1)
  lsems = (sem0, sem1)

  def mk_load(blk, par):
    return pltpu.make_async_copy(
        feat_hbm.at[pl.ds(row0 + blk * BLK, BLK), pl.ds(col0, COLS)],
        fbufs[par], lsems[par])

  mk_load(0, 0).start()
  mk_load(1, 1).start()

  def process(blk, fbuf):
    @plsc.parallel_loop(0, BLK, unroll=4)
    def _rows(r):
      cls_splat = plsc.load_gather(off_v, [zi + (blk * BLK + r)])
      for j in range(CHUNKS):
        x = fbuf[r, pl.ds(j * L, L)]
        plsc.addupdate_scatter(cbuf_v, [cls_splat, iota + j * L], x)

  for blk in range(NBLK):
    par = blk % 2
    mk_load(blk, par).wait()
    process(blk, fbufs[par])
    if blk + 2 < NBLK:
      mk_load(blk + 2, par).start()

  n0 = zf + jnp.sum(cnt_v[0, pl.ds(0, L)])
  n1 = zf + jnp.sum(cnt_v[1, pl.ds(0, L)])
  n2 = zf + jnp.sum(cnt_v[2, pl.ds(0, L)])
  for j in range(CHUNKS):
    cbuf_v[3, pl.ds(j * L, L)] = n0
    cbuf_v[4, pl.ds(j * L, L)] = n1
    cbuf_v[5, pl.ds(j * L, L)] = n2
  pltpu.sync_copy(cbuf_v, shared.at[pl.ds(sid * SROWS, SROWS)])
  plsc.subcore_barrier()

  @pl.when(sid == 0)
  def _finalize():
    pltpu.sync_copy(shared.at[pl.ds(0, SROWS)], rbuf_v)

    def red_body(t, _):
      pltpu.sync_copy(shared.at[pl.ds(t * SROWS, SROWS)], tbuf_v)
      for k in range(6):
        for j in range(CHUNKS):
          rbuf_v[k, pl.ds(j * L, L)] = (
              rbuf_v[k, pl.ds(j * L, L)] + tbuf_v[k, pl.ds(j * L, L)])
      return 0
    lax.fori_loop(1, NS, red_body, 0)

    ones = jnp.ones((L,), jnp.float32)
    for k in range(N_CLS):
      for j in range(CHUNKS):
        cnt = rbuf_v[3 + k, pl.ds(j * L, L)]
        safe = jnp.where(cnt > 0, cnt, ones)
        scale = jnp.where(cnt > 0, ones / safe, ones)
        rbuf_v[k, pl.ds(j * L, L)] = rbuf_v[k, pl.ds(j * L, L)] * scale
    pltpu.sync_copy(rbuf_v.at[pl.ds(0, N_CLS)],
                    out_hbm.at[:, pl.ds(col0, COLS)])


@jax.jit
def kernel(features, pseudo_labels):
  mesh = plsc.VectorSubcoreMesh(core_axis_name="c", subcore_axis_name="s")
  run = pl.kernel(
      _body,
      out_type=jax.ShapeDtypeStruct((N_CLS, D), jnp.float32),
      mesh=mesh,
      compiler_params=pltpu.CompilerParams(needs_layout_passes=False),
      scratch_types=[
          pltpu.VMEM((ROWS * N_CLS,), jnp.float32),
          pltpu.VMEM((ROWS,), jnp.int32),
          pltpu.VMEM((N_CLS, L), jnp.float32),
          pltpu.VMEM((SROWS, COLS), jnp.float32),
          pltpu.VMEM((SROWS, COLS), jnp.float32),
          pltpu.VMEM((SROWS, COLS), jnp.float32),
          pltpu.VMEM((BLK, COLS), jnp.float32),
          pltpu.VMEM((BLK, COLS), jnp.float32),
          pltpu.SemaphoreType.DMA,
          pltpu.SemaphoreType.DMA,
          pltpu.VMEM_SHARED((NS * SROWS, COLS), jnp.float32),
      ],
  )
  return run(features, pseudo_labels.reshape(-1))


# speedup vs baseline: 2.6928x; 1.0448x over previous
import jax
import jax.numpy as jnp
from jax import lax
from jax.experimental import pallas as pl
from jax.experimental.pallas import tpu as pltpu
from jax.experimental.pallas import tpu_sc as plsc

N_CLS = 3
D = 512
B = 16384

NC = 2
NS = 16
L = 16

COLS = D // NC
ROWS = B // NS
BLK = 128
NBLK = ROWS // BLK
CHUNKS = COLS // L
GROUPS = ROWS // L
SROWS = 8


def _body(feat_hbm, lab_hbm, out_hbm,
          lab_v, off_v, cnt_v, cbuf_v, tbuf_v, rbuf_v, fbuf0, fbuf1,
          sem0, sem1, shared):
  cid = lax.axis_index("c")
  sid = lax.axis_index("s")
  row0 = sid * ROWS
  col0 = cid * COLS

  iota = lax.iota(jnp.int32, L)
  zf = jnp.zeros((L,), jnp.float32)
  zi = jnp.zeros((L,), jnp.int32)

  fbufs = (fbuf0, fbuf1)
  lsems = (sem0, sem1)

  def mk_load(blk, par):
    return pltpu.make_async_copy(
        feat_hbm.at[pl.ds(row0 + blk * BLK, BLK), pl.ds(col0, COLS)],
        fbufs[par], lsems[par])

  mk_load(0, 0).start()
  mk_load(1, 1).start()

  pltpu.sync_copy(lab_hbm.at[pl.ds(row0 * N_CLS, ROWS * N_CLS)], lab_v)

  for k in range(6):
    for j in range(CHUNKS):
      cbuf_v[k, pl.ds(j * L, L)] = zf
  for k in range(N_CLS):
    cnt_v[k, pl.ds(0, L)] = zf

  iota3 = iota * 3
  one = jnp.float32(1.0)
  zero = jnp.float32(0.0)

  @plsc.parallel_loop(0, GROUPS, unroll=4)
  def _cls(g):
    base = g * (L * N_CLS)
    p0 = plsc.load_gather(lab_v, [iota3 + base])
    p1 = plsc.load_gather(lab_v, [iota3 + (base + 1)])
    p2 = plsc.load_gather(lab_v, [iota3 + (base + 2)])
    cls = jnp.where(p1 > p0, 1, 0).astype(jnp.int32)
    m01 = jnp.maximum(p0, p1)
    cls = jnp.where(p2 > m01, 2, cls)
    off_v[pl.ds(g * L, L)] = cls
    plsc.addupdate(cnt_v.at[0, pl.ds(0, L)], jnp.where(cls == 0, one, zero))
    plsc.addupdate(cnt_v.at[1, pl.ds(0, L)], jnp.where(cls == 1, one, zero))
    plsc.addupdate(cnt_v.at[2, pl.ds(0, L)], jnp.where(cls == 2, one, zero))

  def process(blk, fbuf):
    @plsc.parallel_loop(0, BLK, unroll=4)
    def _rows(r):
      cls_splat = plsc.load_gather(off_v, [zi + (blk * BLK + r)])
      for j in range(CHUNKS):
        x = fbuf[r, pl.ds(j * L, L)]
        plsc.addupdate_scatter(cbuf_v, [cls_splat, iota + j * L], x)

  for blk in range(NBLK):
    par = blk % 2
    mk_load(blk, par).wait()
    process(blk, fbufs[par])
    if blk + 2 < NBLK:
      mk_load(blk + 2, par).start()

  n0 = zf + jnp.sum(cnt_v[0, pl.ds(0, L)])
  n1 = zf + jnp.sum(cnt_v[1, pl.ds(0, L)])
  n2 = zf + jnp.sum(cnt_v[2, pl.ds(0, L)])
  for j in range(CHUNKS):
    cbuf_v[3, pl.ds(j * L, L)] = n0
    cbuf_v[4, pl.ds(j * L, L)] = n1
    cbuf_v[5, pl.ds(j * L, L)] = n2
  pltpu.sync_copy(cbuf_v, shared.at[pl.ds(sid * SROWS, SROWS)])
  plsc.subcore_barrier()

  for d in (8, 4, 2, 1):
    @pl.when(sid < d)
    def _step(d=d):
      pltpu.sync_copy(shared.at[pl.ds((sid + d) * SROWS, SROWS)], tbuf_v)
      for k in range(6):
        for j in range(CHUNKS):
          cbuf_v[k, pl.ds(j * L, L)] = (
              cbuf_v[k, pl.ds(j * L, L)] + tbuf_v[k, pl.ds(j * L, L)])
      if d > 1:
        pltpu.sync_copy(cbuf_v, shared.at[pl.ds(sid * SROWS, SROWS)])
    plsc.subcore_barrier()

  @pl.when(sid == 0)
  def _finalize():
    ones = jnp.ones((L,), jnp.float32)
    for k in range(N_CLS):
      for j in range(CHUNKS):
        cnt = cbuf_v[3 + k, pl.ds(j * L, L)]
        safe = jnp.where(cnt > 0, cnt, ones)
        scale = jnp.where(cnt > 0, ones / safe, ones)
        cbuf_v[k, pl.ds(j * L, L)] = cbuf_v[k, pl.ds(j * L, L)] * scale
    pltpu.sync_copy(cbuf_v.at[pl.ds(0, N_CLS)],
                    out_hbm.at[:, pl.ds(col0, COLS)])


@jax.jit
def kernel(features, pseudo_labels):
  mesh = plsc.VectorSubcoreMesh(core_axis_name="c", subcore_axis_name="s")
  run = pl.kernel(
      _body,
      out_type=jax.ShapeDtypeStruct((N_CLS, D), jnp.float32),
      mesh=mesh,
      compiler_params=pltpu.CompilerParams(needs_layout_passes=False),
      scratch_types=[
          pltpu.VMEM((ROWS * N_CLS,), jnp.float32),
          pltpu.VMEM((ROWS,), jnp.int32),
          pltpu.VMEM((N_CLS, L), jnp.float32),
          pltpu.VMEM((SROWS, COLS), jnp.float32),
          pltpu.VMEM((SROWS, COLS), jnp.float32),
          pltpu.VMEM((SROWS, COLS), jnp.float32),
          pltpu.VMEM((BLK, COLS), jnp.float32),
          pltpu.VMEM((BLK, COLS), jnp.float32),
          pltpu.SemaphoreType.DMA,
          pltpu.SemaphoreType.DMA,
          pltpu.VMEM_SHARED((NS * SROWS, COLS), jnp.float32),
      ],
  )
  return run(features, pseudo_labels.reshape(-1))
